# baseline (speedup 1.0000x reference)
import functools

import jax
import jax.numpy as jnp
import numpy as np
from jax import lax
from jax.experimental import pallas as pl
from jax.experimental.pallas import tpu as pltpu
from jax.experimental.pallas import tpu_sc as plsc

_NC = 2
_NS = 16
_L = 16
_UNROLL = 8

_TL, _TH = 0.49, 0.565
_KTL = int(np.float32(_TL).view(np.int32))
_KTH = int(np.float32(_TH).view(np.int32))
_IMIN = -(2 ** 31)
_IMAX = 2 ** 31 - 1


def _i32(x):
    return x.astype(jnp.int32)


def _splat(x, dtype=jnp.int32):
    return jnp.full((_L,), x, dtype=dtype)


def _decode(c):
    bits = jnp.where(c >= 0, c, c ^ jnp.int32(0x7FFFFFFF))
    return lax.bitcast_convert_type(bits, jnp.float32)


def _make_sc_kernel(b, n, k):
    rows_per_w = b // (_NC * _NS)
    n_chunks = n // (_L * _UNROLL)
    mesh = plsc.VectorSubcoreMesh(core_axis_name="c", subcore_axis_name="s")

    @functools.partial(
        pl.kernel,
        out_type=jax.ShapeDtypeStruct((b, n), jnp.float32),
        mesh=mesh,
        scratch_types=[
            pltpu.VMEM((n,), jnp.float32),
            pltpu.VMEM((n,), jnp.float32),
            pltpu.VMEM((n + 4 * _L,), jnp.float32),
            pltpu.SemaphoreType.DMA,
            pltpu.SemaphoreType.DMA,
            pltpu.SemaphoreType.DMA,
            pltpu.SemaphoreType.DMA,
        ],
        compiler_params=pltpu.CompilerParams(needs_layout_passes=False),
    )
    def sc_kernel(x_hbm, out_hbm, row_a, row_b, cbuf_v,
                  sem_in_a, sem_in_b, sem_out_a, sem_out_b):
        wid = lax.axis_index("s") * _NC + lax.axis_index("c")
        lane = lax.iota(jnp.int32, _L)
        zero_i = _splat(0)
        kk = jnp.int32(k)
        tls = _splat(_TL, jnp.float32)
        ths = _splat(_TH, jnp.float32)

        def process_row(row_v, row):
            def pa_body(i, carry):
                a_hi, off_vec = carry
                xv = row_v[pl.ds(i * _L, _L)]
                ge_hi = xv >= ths
                a_hi = a_hi + _i32(ge_hi)
                keep = (xv >= tls) & (xv < ths)
                idx = lane + (off_vec << 4)
                plsc.store_scatter(cbuf_v, [idx], xv, mask=keep)
                off_vec = off_vec + _i32(keep)
                return a_hi, off_vec

            a_hi, off_vec = plsc.parallel_loop(
                0, n // _L, unroll=16, carry=(zero_i, zero_i))(pa_body)
            n_hi = jnp.sum(a_hi)
            n_lo = n_hi + jnp.sum(off_vec)
            max_off = jnp.max(off_vec)

            in_window = (n_hi < kk) & (n_lo >= kk)

            def fast_fn():
                def cond(carry):
                    lo, hi = carry
                    return (hi - lo) != 1

                def body(carry):
                    lo, hi = carry
                    c = lo + lax.shift_right_logical(hi - lo, 1)
                    ts = _splat(_decode(c), jnp.float32)

                    def bcnt(j, acc):
                        xv = cbuf_v[pl.ds(j * _L, _L)]
                        valid = _splat(j) < off_vec
                        return acc + _i32(valid & (xv >= ts))

                    acc = plsc.parallel_loop(
                        0, max_off, unroll=4, carry=zero_i)(bcnt)
                    cnt = n_hi + jnp.sum(acc)
                    ok = cnt >= kk
                    return jnp.where(ok, c, lo), jnp.where(ok, hi, c)

                lo, _ = lax.while_loop(cond, body,
                                       (jnp.int32(_KTL), jnp.int32(_KTH)))
                return lo

            def slow_fn():
                below = n_lo < kk
                lo0 = jnp.where(below, jnp.int32(_IMIN), jnp.int32(_KTH))
                hi0 = jnp.where(below, jnp.int32(_KTL), jnp.int32(_IMAX))

                def cond(carry):
                    lo, hi = carry
                    return (hi - lo) != 1

                def body(carry):
                    lo, hi = carry
                    c = lo + lax.shift_right_logical(hi - lo, 1)
                    ts = _splat(_decode(c), jnp.float32)

                    def bcnt(i, acc):
                        base = i * (_L * _UNROLL)
                        for u in range(_UNROLL):
                            xv = row_v[pl.ds(base + u * _L, _L)]
                            acc = acc + _i32(xv >= ts)
                        return acc

                    acc = lax.fori_loop(0, n_chunks, bcnt, zero_i)
                    cnt = jnp.sum(acc)
                    ok = cnt >= kk
                    return jnp.where(ok, c, lo), jnp.where(ok, hi, c)

                lo, _ = lax.while_loop(cond, body, (lo0, hi0))
                return lo

            tkey = lax.cond(in_window, fast_fn, slow_fn)
            tf = _splat(_decode(tkey), jnp.float32)

            one_f = _splat(1.0, jnp.float32)
            zero_f = _splat(0.0, jnp.float32)

            def mask_body(i):
                sl = pl.ds(i * _L, _L)
                row_v[sl] = jnp.where(row_v[sl] >= tf, one_f, zero_f)

            plsc.parallel_loop(0, n // _L, unroll=_UNROLL)(mask_body)

        bufs = [row_a, row_b]
        sem_in = [sem_in_a, sem_in_b]
        sem_out = [sem_out_a, sem_out_b]
        base_row = wid * rows_per_w
        h_in = [pltpu.async_copy(x_hbm.at[base_row + p], bufs[p], sem_in[p])
                for p in range(2)]
        h_out = [None, None]
        for r in range(rows_per_w):
            p = r % 2
            h_in[p].wait()
            process_row(bufs[p], base_row + r)
            if h_out[p] is not None:
                h_out[p].wait()
            h_out[p] = pltpu.async_copy(bufs[p], out_hbm.at[base_row + r],
                                        sem_out[p])
            if r + 2 < rows_per_w:
                h_out[p].wait()
                h_out[p] = None
                h_in[p] = pltpu.async_copy(x_hbm.at[base_row + r + 2],
                                           bufs[p], sem_in[p])
        for p in range(2):
            if h_out[p] is not None:
                h_out[p].wait()

    return sc_kernel


@jax.jit
def kernel(attention_scores):
    b, n = attention_scores.shape
    k = max(1, int(n * 0.3))
    return _make_sc_kernel(b, n, k)(attention_scores)


# speedup vs baseline: 1.4560x; 1.4560x over previous
import functools

import jax
import jax.numpy as jnp
import numpy as np
from jax import lax
from jax.experimental import pallas as pl
from jax.experimental.pallas import tpu as pltpu
from jax.experimental.pallas import tpu_sc as plsc

_NC = 2
_NS = 16
_L = 16
_UNROLL = 8

_TL, _TH = 0.49, 0.565
_KTL = int(np.float32(_TL).view(np.int32))
_KTH = int(np.float32(_TH).view(np.int32))
_IMIN = -(2 ** 31)
_IMAX = 2 ** 31 - 1


def _i32(x):
    return x.astype(jnp.int32)


def _splat(x, dtype=jnp.int32):
    return jnp.full((_L,), x, dtype=dtype)


def _decode(c):
    bits = jnp.where(c >= 0, c, c ^ jnp.int32(0x7FFFFFFF))
    return lax.bitcast_convert_type(bits, jnp.float32)


def _make_sc_kernel(b, n, k):
    rows_per_w = b // (_NC * _NS)
    n_chunks = n // (_L * _UNROLL)
    mesh = plsc.VectorSubcoreMesh(core_axis_name="c", subcore_axis_name="s")

    @functools.partial(
        pl.kernel,
        out_type=jax.ShapeDtypeStruct((b, n), jnp.float32),
        mesh=mesh,
        scratch_types=[
            pltpu.VMEM((n,), jnp.float32),
            pltpu.VMEM((n,), jnp.float32),
            pltpu.VMEM((n + 4 * _L,), jnp.float32),
            pltpu.SemaphoreType.DMA,
            pltpu.SemaphoreType.DMA,
            pltpu.SemaphoreType.DMA,
            pltpu.SemaphoreType.DMA,
        ],
        compiler_params=pltpu.CompilerParams(needs_layout_passes=False),
    )
    def sc_kernel(x_hbm, out_hbm, row_a, row_b, cbuf_v,
                  sem_in_a, sem_in_b, sem_out_a, sem_out_b):
        wid = lax.axis_index("s") * _NC + lax.axis_index("c")
        lane = lax.iota(jnp.int32, _L)
        zero_i = _splat(0)
        kk = jnp.int32(k)
        tls = _splat(_TL, jnp.float32)
        ths = _splat(_TH, jnp.float32)

        def process_row(row_v, row):
            def pa_body(i, carry):
                a_hi, off_vec = carry
                xv = row_v[pl.ds(i * _L, _L)]
                ge_hi = xv >= ths
                a_hi = a_hi + _i32(ge_hi)
                keep = (xv >= tls) & (xv < ths)
                idx = lane + (off_vec << 4)
                plsc.store_scatter(cbuf_v, [idx], xv, mask=keep)
                off_vec = off_vec + _i32(keep)
                return a_hi, off_vec

            a_hi, off_vec = plsc.parallel_loop(
                0, n // _L, unroll=_UNROLL, carry=(zero_i, zero_i))(pa_body)
            n_hi = jnp.sum(a_hi)
            n_lo = n_hi + jnp.sum(off_vec)
            max_off = jnp.max(off_vec)

            in_window = (n_hi < kk) & (n_lo >= kk)

            def fast_fn():
                def cond(carry):
                    lo, hi = carry
                    return (hi - lo) != 1

                def body(carry):
                    lo, hi = carry
                    c = lo + lax.shift_right_logical(hi - lo, 1)
                    ts = _splat(_decode(c), jnp.float32)

                    def bcnt(j, acc):
                        xv = cbuf_v[pl.ds(j * _L, _L)]
                        valid = _splat(j) < off_vec
                        return acc + _i32(valid & (xv >= ts))

                    acc = plsc.parallel_loop(
                        0, max_off, unroll=4, carry=zero_i)(bcnt)
                    cnt = n_hi + jnp.sum(acc)
                    ok = cnt >= kk
                    return jnp.where(ok, c, lo), jnp.where(ok, hi, c)

                lo, _ = lax.while_loop(cond, body,
                                       (jnp.int32(_KTL), jnp.int32(_KTH)))
                return lo

            def slow_fn():
                below = n_lo < kk
                lo0 = jnp.where(below, jnp.int32(_IMIN), jnp.int32(_KTH))
                hi0 = jnp.where(below, jnp.int32(_KTL), jnp.int32(_IMAX))

                def cond(carry):
                    lo, hi = carry
                    return (hi - lo) != 1

                def body(carry):
                    lo, hi = carry
                    c = lo + lax.shift_right_logical(hi - lo, 1)
                    ts = _splat(_decode(c), jnp.float32)

                    def bcnt(i, acc):
                        base = i * (_L * _UNROLL)
                        for u in range(_UNROLL):
                            xv = row_v[pl.ds(base + u * _L, _L)]
                            acc = acc + _i32(xv >= ts)
                        return acc

                    acc = lax.fori_loop(0, n_chunks, bcnt, zero_i)
                    cnt = jnp.sum(acc)
                    ok = cnt >= kk
                    return jnp.where(ok, c, lo), jnp.where(ok, hi, c)

                lo, _ = lax.while_loop(cond, body, (lo0, hi0))
                return lo

            tkey = lax.cond(in_window, fast_fn, slow_fn)
            tf = _splat(_decode(tkey), jnp.float32)

            one_f = _splat(1.0, jnp.float32)
            zero_f = _splat(0.0, jnp.float32)

            def mask_body(i):
                sl = pl.ds(i * _L, _L)
                row_v[sl] = jnp.where(row_v[sl] >= tf, one_f, zero_f)

            plsc.parallel_loop(0, n // _L, unroll=_UNROLL)(mask_body)

        bufs = [row_a, row_b]
        sem_in = [sem_in_a, sem_in_b]
        sem_out = [sem_out_a, sem_out_b]
        base_row = wid * rows_per_w
        h_in = [pltpu.async_copy(x_hbm.at[base_row + p], bufs[p], sem_in[p])
                for p in range(2)]
        h_out = [None, None]
        for r in range(rows_per_w):
            p = r % 2
            h_in[p].wait()
            process_row(bufs[p], base_row + r)
            if h_out[p] is not None:
                h_out[p].wait()
            h_out[p] = pltpu.async_copy(bufs[p], out_hbm.at[base_row + r],
                                        sem_out[p])
            if r + 2 < rows_per_w:
                h_out[p].wait()
                h_out[p] = None
                h_in[p] = pltpu.async_copy(x_hbm.at[base_row + r + 2],
                                           bufs[p], sem_in[p])
        for p in range(2):
            if h_out[p] is not None:
                h_out[p].wait()

    return sc_kernel


@jax.jit
def kernel(attention_scores):
    b, n = attention_scores.shape
    k = max(1, int(n * 0.3))
    return _make_sc_kernel(b, n, k)(attention_scores)
